# baseline (speedup 1.0000x reference)
import functools

import jax
import jax.numpy as jnp
from jax import lax
from jax.experimental import pallas as pl
from jax.experimental.pallas import tpu as pltpu
from jax.experimental.pallas import tpu_sc as plsc

N = 10000
D = 128
E = 320000

NW = 32
NB = 320
NBA = NB + 8
NPAD = NW * NB
C = 6400
NCH = E // C
NPAIR = NCH // 2
NG = C // 16
G = 128
EMAX = 160 * 2048
SEG = 2048

_MESH = plsc.VectorSubcoreMesh(
    core_axis_name="c", subcore_axis_name="s", num_cores=2, num_subcores=16)
_CP = pltpu.CompilerParams(needs_layout_passes=False)


def _init_acc(acc):
    def init_acc(i, _):
        row = acc.at[i]
        for r in range(D // 16):
            row[pl.ds(r * 16, 16)] = jnp.full((16,), -jnp.inf, jnp.float32)
        return 0
    lax.fori_loop(0, NBA, init_acc, 0)


def _process_block(total, b, rbuf, dref, dref_off, acc):
    bbase = b * G

    def edge(e, _):
        rel = e - bbase
        ldst = dref[pl.ds(dref_off + e, 16)][0]
        arow = acc.at[ldst]
        rrow = rbuf.at[rel]
        for r in range(D // 16):
            a = arow[pl.ds(r * 16, 16)]
            v = rrow[pl.ds(r * 16, 16)]
            arow[pl.ds(r * 16, 16)] = jnp.maximum(a, v)
        return 0
    lax.fori_loop(bbase, jnp.minimum(total, bbase + G), edge, 0)


def _gather_and_process(total, sref, sref_off, dref, dref_off,
                        h_hbm, rows0, rows1, sg0, sg1, acc):
    nblk = (total + (G - 1)) // G

    def blk(b, _):
        pltpu.async_copy(h_hbm.at[sref.at[pl.ds(sref_off + b * G, G)]],
                         rows0, sg0).wait()
        _process_block(total, b, rows0, dref, dref_off, acc)
        return 0
    lax.fori_loop(0, nblk, blk, 0)


def _seg_scan_body(h_hbm, src_hbm, dst_hbm, out_hbm,
                   dstv, srcv, sbuf, dbuf, stage, rows0, rows1, acc,
                   sd0, sd1, sg0, sg1, sf):
    w = lax.axis_index("s") * 2 + lax.axis_index("c")
    lo = w * NB
    hi = jnp.minimum(lo + NB, N)
    lbase = w * EMAX

    _init_acc(acc)

    def init_b(i, _):
        sbuf[pl.ds(i * 16, 16)] = jnp.zeros((16,), jnp.int32)
        return 0
    lax.fori_loop(0, (C + 144) // 16, init_b, 0)

    def fire(cbase, boff, sem):
        pltpu.async_copy(dst_hbm.at[pl.ds(cbase, C)],
                         dstv.at[pl.ds(boff, C)], sem)
        pltpu.async_copy(src_hbm.at[pl.ds(cbase, C)],
                         srcv.at[pl.ds(boff, C)], sem)

    def wait(cbase, boff, sem):
        pltpu.make_async_copy(dst_hbm.at[pl.ds(cbase, C)],
                              dstv.at[pl.ds(boff, C)], sem).wait()
        pltpu.make_async_copy(src_hbm.at[pl.ds(cbase, C)],
                              srcv.at[pl.ds(boff, C)], sem).wait()

    def scan_process(boff, gpos):
        def scan_g(g, off):
            d = dstv[pl.ds(boff + g * 16, 16)]
            s = srcv[pl.ds(boff + g * 16, 16)]
            m = (d >= lo) & (d < hi)
            pos = off + plsc.cumsum(m.astype(jnp.int32)) - 1
            plsc.store_scatter(sbuf, [pos], s, mask=m)
            plsc.store_scatter(dbuf, [pos], d - lo, mask=m)
            return off + plsc.all_reduce_population_count(m)[0]
        total = lax.fori_loop(0, NG, scan_g, jnp.int32(0))

        io16 = lax.iota(jnp.int32, 16)
        for k in range(8):
            spos = total + k * 16 + io16
            plsc.store_scatter(sbuf, [spos], jnp.zeros((16,), jnp.int32))
            plsc.store_scatter(dbuf, [spos], jnp.full((16,), NB, jnp.int32))
        nf = (total + 127) // 128

        _gather_and_process(total, sbuf, 0, dbuf, 0,
                            h_hbm, rows0, rows1, sg0, sg1, acc)

        return gpos + nf * 128

    fire(0, 0, sd0)

    def pair(p, gpos):
        c0 = 2 * p * C
        c1 = c0 + C
        fire(c1, C, sd1)
        wait(c0, 0, sd0)
        gpos = scan_process(0, gpos)

        @pl.when(p + 1 < NPAIR)
        def _():
            fire(c0 + 2 * C, 0, sd0)
        wait(c1, C, sd1)
        gpos = scan_process(C, gpos)
        return gpos
    gpos = lax.fori_loop(0, NPAIR, pair, jnp.int32(0))

    pltpu.sync_copy(acc.at[pl.ds(0, NB)], out_hbm.at[pl.ds(lo, NB)])


@functools.partial(
    pl.kernel,
    out_type=jax.ShapeDtypeStruct((NPAD, D), jnp.float32),
    mesh=_MESH,
    compiler_params=_CP,
    scratch_types=[
        pltpu.VMEM((2 * C,), jnp.int32),
        pltpu.VMEM((2 * C,), jnp.int32),
        pltpu.VMEM((C + 144,), jnp.int32),
        pltpu.VMEM((C + 144,), jnp.int32),
        pltpu.VMEM((16,), jnp.int32),
        pltpu.VMEM((G, D), jnp.float32),
        pltpu.VMEM((G, D), jnp.float32),
        pltpu.VMEM((NBA, D), jnp.float32),
        pltpu.SemaphoreType.DMA,
        pltpu.SemaphoreType.DMA,
        pltpu.SemaphoreType.DMA,
        pltpu.SemaphoreType.DMA,
        pltpu.SemaphoreType.DMA,
    ],
)
def _seg_scan(h_hbm, src_hbm, dst_hbm, out_hbm,
              dstv, srcv, sbuf, dbuf, stage, rows0, rows1, acc,
              sd0, sd1, sg0, sg1, sf):
    _seg_scan_body(h_hbm, src_hbm, dst_hbm, out_hbm,
                   dstv, srcv, sbuf, dbuf, stage, rows0, rows1, acc,
                   sd0, sd1, sg0, sg1, sf)


def _seg_replay_body(h_hbm, sl_hbm, dl_hbm, tot_hbm, out_hbm,
                     segs, segd, stage, rows0, rows1, acc,
                     ss0, ss1, sg0, sg1):
    w = lax.axis_index("s") * 2 + lax.axis_index("c")
    lo = w * NB
    lbase = w * EMAX

    pltpu.sync_copy(tot_hbm.at[pl.ds(w * 16, 16)], stage)
    T = stage[pl.ds(0, 16)][0]

    _init_acc(acc)

    nseg = (T + SEG - 1) // SEG

    def sfire(i, par, sem):
        pltpu.async_copy(sl_hbm.at[pl.ds(lbase + i * SEG, SEG)],
                         segs.at[pl.ds(par * SEG, SEG)], sem)
        pltpu.async_copy(dl_hbm.at[pl.ds(lbase + i * SEG, SEG)],
                         segd.at[pl.ds(par * SEG, SEG)], sem)

    def swait(i, par, sem):
        pltpu.make_async_copy(sl_hbm.at[pl.ds(lbase + i * SEG, SEG)],
                              segs.at[pl.ds(par * SEG, SEG)], sem).wait()
        pltpu.make_async_copy(dl_hbm.at[pl.ds(lbase + i * SEG, SEG)],
                              segd.at[pl.ds(par * SEG, SEG)], sem).wait()

    def proc_seg(i, par):
        cnt = jnp.minimum(T - i * SEG, SEG)
        _gather_and_process(cnt, segs, par * SEG, segd, par * SEG,
                            h_hbm, rows0, rows1, sg0, sg1, acc)

    @pl.when(nseg > 0)
    def _():
        sfire(0, 0, ss0)

    def spair(p, _):
        i0 = 2 * p
        i1 = i0 + 1

        @pl.when(i1 < nseg)
        def _():
            sfire(i1, 1, ss1)
        swait(i0, 0, ss0)
        proc_seg(i0, 0)

        @pl.when(i1 + 1 < nseg)
        def _():
            sfire(i1 + 1, 0, ss0)

        @pl.when(i1 < nseg)
        def _():
            swait(i1, 1, ss1)
            proc_seg(i1, 1)
        return 0
    lax.fori_loop(0, (nseg + 1) // 2, spair, 0)

    pltpu.sync_copy(acc.at[pl.ds(0, NB)], out_hbm.at[pl.ds(lo, NB)])


@functools.partial(
    pl.kernel,
    out_type=jax.ShapeDtypeStruct((NPAD, D), jnp.float32),
    mesh=_MESH,
    compiler_params=_CP,
    scratch_types=[
        pltpu.VMEM((2 * SEG,), jnp.int32),
        pltpu.VMEM((2 * SEG + 16,), jnp.int32),
        pltpu.VMEM((16,), jnp.int32),
        pltpu.VMEM((G, D), jnp.float32),
        pltpu.VMEM((G, D), jnp.float32),
        pltpu.VMEM((NBA, D), jnp.float32),
        pltpu.SemaphoreType.DMA,
        pltpu.SemaphoreType.DMA,
        pltpu.SemaphoreType.DMA,
        pltpu.SemaphoreType.DMA,
    ],
)
def _seg_replay(h_hbm, sl_hbm, dl_hbm, tot_hbm, out_hbm,
                segs, segd, stage, rows0, rows1, acc, ss0, ss1, sg0, sg1):
    _seg_replay_body(h_hbm, sl_hbm, dl_hbm, tot_hbm, out_hbm,
                     segs, segd, stage, rows0, rows1, acc, ss0, ss1, sg0, sg1)


BR = 2000


def _tc_layer_kernel(h_ref, a_ref, wl_ref, bl_ref, wr_ref, o_ref):
    a = a_ref[...]
    a = jnp.where(jnp.isfinite(a), a, 0.0)
    o = jnp.dot(a, wl_ref[...], preferred_element_type=jnp.float32)
    o = o + jnp.dot(h_ref[...], wr_ref[...], preferred_element_type=jnp.float32)
    o = o + bl_ref[...]
    o_ref[...] = jnp.maximum(o, 0.0)


def _tc_layer(h, aggp, wlt, bl, wrt):
    return pl.pallas_call(
        _tc_layer_kernel,
        grid=(N // BR,),
        in_specs=[
            pl.BlockSpec((BR, D), lambda i: (i, 0)),
            pl.BlockSpec((BR, D), lambda i: (i, 0)),
            pl.BlockSpec((D, D), lambda i: (0, 0)),
            pl.BlockSpec((1, D), lambda i: (0, 0)),
            pl.BlockSpec((D, D), lambda i: (0, 0)),
        ],
        out_specs=pl.BlockSpec((BR, D), lambda i: (i, 0)),
        out_shape=jax.ShapeDtypeStruct((N, D), jnp.float32),
    )(h, aggp, wlt, bl, wrt)


def _tc_final_kernel(h_ref, wt_ref, b_ref, q_ref, o_ref):
    g = jnp.dot(h_ref[...], wt_ref[...], preferred_element_type=jnp.float32)
    g = g + b_ref[...]
    q = q_ref[...]
    qn = jnp.sqrt(jnp.sum(q * q))
    nrm = jnp.sqrt(jnp.sum(g * g, axis=1, keepdims=True))
    s = jnp.dot(g, q.T, preferred_element_type=jnp.float32)
    o_ref[...] = s / (jnp.maximum(nrm, 1e-12) * jnp.maximum(qn, 1e-12))


def _tc_final(h, wlint, blin, q):
    return pl.pallas_call(
        _tc_final_kernel,
        grid=(N // BR,),
        in_specs=[
            pl.BlockSpec((BR, D), lambda i: (i, 0)),
            pl.BlockSpec((D, D), lambda i: (0, 0)),
            pl.BlockSpec((1, D), lambda i: (0, 0)),
            pl.BlockSpec((1, D), lambda i: (0, 0)),
        ],
        out_specs=pl.BlockSpec((BR, 1), lambda i: (i, 0)),
        out_shape=jax.ShapeDtypeStruct((N, 1), jnp.float32),
    )(h, wlint, blin, q)


def kernel(x, edge_index, query,
           W_l0, b_l0, W_r0, W_l1, b_l1, W_r1, W_l2, b_l2, W_r2,
           W_lin, b_lin):
    src = edge_index[0].astype(jnp.int32)
    dst = edge_index[1].astype(jnp.int32)
    aggp = _seg_scan(x, src, dst)
    h = _tc_layer(x, aggp[:N], W_l0.T, b_l0.reshape(1, D), W_r0.T)
    for (wl, bl, wr) in [(W_l1, b_l1, W_r1), (W_l2, b_l2, W_r2)]:
        aggp = _seg_scan(h, src, dst)
        h = _tc_layer(h, aggp[:N], wl.T, bl.reshape(1, D), wr.T)
    scores = _tc_final(h, W_lin.T, b_lin.reshape(1, D), query.reshape(1, D))
    return scores[:, 0]


# speedup vs baseline: 2.2967x; 2.2967x over previous
import functools

import jax
import jax.numpy as jnp
from jax import lax
from jax.experimental import pallas as pl
from jax.experimental.pallas import tpu as pltpu
from jax.experimental.pallas import tpu_sc as plsc

N = 10000
D = 128
E = 320000

NW = 32
NB = 320
NPAD = NW * NB
C = 10000
NCH = E // C
NPAIR = NCH // 2
NG = C // 16
G = 128

_MESH = plsc.VectorSubcoreMesh(
    core_axis_name="c", subcore_axis_name="s", num_cores=2, num_subcores=16)


def _seg_max_body(h_hbm, src_hbm, dst_hbm, out_hbm,
                  dstv, srcv, sbuf, dbuf, rows0, acc, sd0, sd1, sg0):
    w = lax.axis_index("s") * 2 + lax.axis_index("c")
    lo = w * NB
    hi = jnp.minimum(lo + NB, N)

    def init_acc(i, _):
        row = acc.at[i]
        for r in range(D // 16):
            row[pl.ds(r * 16, 16)] = jnp.full((16,), -jnp.inf, jnp.float32)
        return 0
    lax.fori_loop(0, NB, init_acc, 0)

    def init_b(i, _):
        sbuf[pl.ds(i * 16, 16)] = jnp.zeros((16,), jnp.int32)
        return 0
    lax.fori_loop(0, (C + 16) // 16, init_b, 0)

    def fire(cbase, boff, sem):
        pltpu.async_copy(dst_hbm.at[pl.ds(cbase, C)],
                         dstv.at[pl.ds(boff, C)], sem)
        pltpu.async_copy(src_hbm.at[pl.ds(cbase, C)],
                         srcv.at[pl.ds(boff, C)], sem)

    def wait(cbase, boff, sem):
        pltpu.make_async_copy(dst_hbm.at[pl.ds(cbase, C)],
                              dstv.at[pl.ds(boff, C)], sem).wait()
        pltpu.make_async_copy(src_hbm.at[pl.ds(cbase, C)],
                              srcv.at[pl.ds(boff, C)], sem).wait()

    def scan_process(boff):
        def scan_g(g, off):
            d = dstv[pl.ds(boff + g * 16, 16)]
            s = srcv[pl.ds(boff + g * 16, 16)]
            m = (d >= lo) & (d < hi)
            pos = off + plsc.cumsum(m.astype(jnp.int32)) - 1
            plsc.store_scatter(sbuf, [pos], s, mask=m)
            plsc.store_scatter(dbuf, [pos], d - lo, mask=m)
            return off + plsc.all_reduce_population_count(m)[0]
        total = lax.fori_loop(0, NG, scan_g, jnp.int32(0))

        nblk = (total + (G - 1)) // G

        def blk(b, _):
            bbase = b * G
            pltpu.async_copy(h_hbm.at[sbuf.at[pl.ds(bbase, G)]],
                             rows0, sg0).wait()

            def edge(e, _):
                rel = e - bbase
                ldst = dbuf[pl.ds(e, 16)][0]
                arow = acc.at[ldst]
                rrow = rows0.at[rel]
                for r in range(D // 16):
                    a = arow[pl.ds(r * 16, 16)]
                    v = rrow[pl.ds(r * 16, 16)]
                    arow[pl.ds(r * 16, 16)] = jnp.maximum(a, v)
                return 0
            lax.fori_loop(bbase, jnp.minimum(total, bbase + G), edge, 0)
            return 0
        lax.fori_loop(0, nblk, blk, 0)

    fire(0, 0, sd0)

    def pair(p, _):
        c0 = 2 * p * C
        c1 = c0 + C
        fire(c1, C, sd1)
        wait(c0, 0, sd0)
        scan_process(0)

        @pl.when(p + 1 < NPAIR)
        def _():
            fire(c0 + 2 * C, 0, sd0)
        wait(c1, C, sd1)
        scan_process(C)
        return 0
    lax.fori_loop(0, NPAIR, pair, 0)

    pltpu.sync_copy(acc, out_hbm.at[pl.ds(lo, NB)])


@functools.partial(
    pl.kernel,
    out_type=jax.ShapeDtypeStruct((NPAD, D), jnp.float32),
    mesh=_MESH,
    compiler_params=pltpu.CompilerParams(needs_layout_passes=False),
    scratch_types=[
        pltpu.VMEM((2 * C,), jnp.int32),
        pltpu.VMEM((2 * C,), jnp.int32),
        pltpu.VMEM((C + 16,), jnp.int32),
        pltpu.VMEM((C + 16,), jnp.int32),
        pltpu.VMEM((G, D), jnp.float32),
        pltpu.VMEM((NB, D), jnp.float32),
        pltpu.SemaphoreType.DMA,
        pltpu.SemaphoreType.DMA,
        pltpu.SemaphoreType.DMA,
    ],
)
def _seg_max(h_hbm, src_hbm, dst_hbm, out_hbm,
             dstv, srcv, sbuf, dbuf, rows0, acc, sd0, sd1, sg0):
    _seg_max_body(h_hbm, src_hbm, dst_hbm, out_hbm,
                  dstv, srcv, sbuf, dbuf, rows0, acc, sd0, sd1, sg0)


BR = 2000


def _tc_layer_kernel(h_ref, a_ref, wl_ref, bl_ref, wr_ref, o_ref):
    a = a_ref[...]
    a = jnp.where(jnp.isfinite(a), a, 0.0)
    o = jnp.dot(a, wl_ref[...], preferred_element_type=jnp.float32)
    o = o + jnp.dot(h_ref[...], wr_ref[...], preferred_element_type=jnp.float32)
    o = o + bl_ref[...]
    o_ref[...] = jnp.maximum(o, 0.0)


def _tc_layer(h, aggp, wlt, bl, wrt):
    return pl.pallas_call(
        _tc_layer_kernel,
        grid=(N // BR,),
        in_specs=[
            pl.BlockSpec((BR, D), lambda i: (i, 0)),
            pl.BlockSpec((BR, D), lambda i: (i, 0)),
            pl.BlockSpec((D, D), lambda i: (0, 0)),
            pl.BlockSpec((1, D), lambda i: (0, 0)),
            pl.BlockSpec((D, D), lambda i: (0, 0)),
        ],
        out_specs=pl.BlockSpec((BR, D), lambda i: (i, 0)),
        out_shape=jax.ShapeDtypeStruct((N, D), jnp.float32),
    )(h, aggp, wlt, bl, wrt)


def _tc_final_kernel(h_ref, wt_ref, b_ref, q_ref, o_ref):
    g = jnp.dot(h_ref[...], wt_ref[...], preferred_element_type=jnp.float32)
    g = g + b_ref[...]
    q = q_ref[...]
    qn = jnp.sqrt(jnp.sum(q * q))
    nrm = jnp.sqrt(jnp.sum(g * g, axis=1, keepdims=True))
    s = jnp.dot(g, q.T, preferred_element_type=jnp.float32)
    o_ref[...] = s / (jnp.maximum(nrm, 1e-12) * jnp.maximum(qn, 1e-12))


def _tc_final(h, wlint, blin, q):
    return pl.pallas_call(
        _tc_final_kernel,
        grid=(N // BR,),
        in_specs=[
            pl.BlockSpec((BR, D), lambda i: (i, 0)),
            pl.BlockSpec((D, D), lambda i: (0, 0)),
            pl.BlockSpec((1, D), lambda i: (0, 0)),
            pl.BlockSpec((1, D), lambda i: (0, 0)),
        ],
        out_specs=pl.BlockSpec((BR, 1), lambda i: (i, 0)),
        out_shape=jax.ShapeDtypeStruct((N, 1), jnp.float32),
    )(h, wlint, blin, q)


def kernel(x, edge_index, query,
           W_l0, b_l0, W_r0, W_l1, b_l1, W_r1, W_l2, b_l2, W_r2,
           W_lin, b_lin):
    src = edge_index[0].astype(jnp.int32)
    dst = edge_index[1].astype(jnp.int32)
    params = [(W_l0, b_l0, W_r0), (W_l1, b_l1, W_r1), (W_l2, b_l2, W_r2)]
    h = x
    for (wl, bl, wr) in params:
        aggp = _seg_max(h, src, dst)
        h = _tc_layer(h, aggp[:N], wl.T, bl.reshape(1, D), wr.T)
    scores = _tc_final(h, W_lin.T, b_lin.reshape(1, D), query.reshape(1, D))
    return scores[:, 0]


# speedup vs baseline: 2.4804x; 1.0800x over previous
import functools

import jax
import jax.numpy as jnp
from jax import lax
from jax.experimental import pallas as pl
from jax.experimental.pallas import tpu as pltpu
from jax.experimental.pallas import tpu_sc as plsc

N = 10000
D = 128
E = 320000

NW = 32
NB = 320
NPAD = NW * NB
C = 16000
NCH = E // C
NPAIR = NCH // 2
NG = C // 16
G = 128

_MESH = plsc.VectorSubcoreMesh(
    core_axis_name="c", subcore_axis_name="s", num_cores=2, num_subcores=16)


def _seg_max_body(h_hbm, src_hbm, dst_hbm, out_hbm,
                  dstv, srcv, sbuf, dbuf, rows0, acc, sd0, sd1, sg0):
    w = lax.axis_index("s") * 2 + lax.axis_index("c")
    lo = w * NB
    hi = jnp.minimum(lo + NB, N)

    def init_acc(i, _):
        row = acc.at[i]
        for r in range(D // 16):
            row[pl.ds(r * 16, 16)] = jnp.full((16,), -jnp.inf, jnp.float32)
        return 0
    lax.fori_loop(0, NB, init_acc, 0)

    def init_b(i, _):
        sbuf[pl.ds(i * 16, 16)] = jnp.zeros((16,), jnp.int32)
        return 0
    lax.fori_loop(0, (C + 16) // 16, init_b, 0)

    def fire(cbase, boff, sem):
        pltpu.async_copy(dst_hbm.at[pl.ds(cbase, C)],
                         dstv.at[pl.ds(boff, C)], sem)
        pltpu.async_copy(src_hbm.at[pl.ds(cbase, C)],
                         srcv.at[pl.ds(boff, C)], sem)

    def wait(cbase, boff, sem):
        pltpu.make_async_copy(dst_hbm.at[pl.ds(cbase, C)],
                              dstv.at[pl.ds(boff, C)], sem).wait()
        pltpu.make_async_copy(src_hbm.at[pl.ds(cbase, C)],
                              srcv.at[pl.ds(boff, C)], sem).wait()

    def scan_process(boff):
        def scan_g(g, off):
            d = dstv[pl.ds(boff + g * 16, 16)]
            s = srcv[pl.ds(boff + g * 16, 16)]
            m = (d >= lo) & (d < hi)
            pos = off + plsc.cumsum(m.astype(jnp.int32)) - 1
            plsc.store_scatter(sbuf, [pos], s, mask=m)
            plsc.store_scatter(dbuf, [pos], d - lo, mask=m)
            return off + plsc.all_reduce_population_count(m)[0]
        total = lax.fori_loop(0, NG, scan_g, jnp.int32(0))

        nblk = (total + (G - 1)) // G

        def blk(b, _):
            bbase = b * G
            pltpu.async_copy(h_hbm.at[sbuf.at[pl.ds(bbase, G)]],
                             rows0, sg0).wait()

            def edge(e, _):
                rel = e - bbase
                ldst = dbuf[pl.ds(e, 16)][0]
                arow = acc.at[ldst]
                rrow = rows0.at[rel]
                for r in range(D // 16):
                    a = arow[pl.ds(r * 16, 16)]
                    v = rrow[pl.ds(r * 16, 16)]
                    arow[pl.ds(r * 16, 16)] = jnp.maximum(a, v)
                return 0
            lax.fori_loop(bbase, jnp.minimum(total, bbase + G), edge, 0)
            return 0
        lax.fori_loop(0, nblk, blk, 0)

    def chunk(c, _):
        fire(c * C, 0, sd0)
        wait(c * C, 0, sd0)
        scan_process(0)
        return 0
    lax.fori_loop(0, NCH, chunk, 0)

    pltpu.sync_copy(acc, out_hbm.at[pl.ds(lo, NB)])


@functools.partial(
    pl.kernel,
    out_type=jax.ShapeDtypeStruct((NPAD, D), jnp.float32),
    mesh=_MESH,
    compiler_params=pltpu.CompilerParams(needs_layout_passes=False),
    scratch_types=[
        pltpu.VMEM((C,), jnp.int32),
        pltpu.VMEM((C,), jnp.int32),
        pltpu.VMEM((C + 16,), jnp.int32),
        pltpu.VMEM((C + 16,), jnp.int32),
        pltpu.VMEM((G, D), jnp.float32),
        pltpu.VMEM((NB, D), jnp.float32),
        pltpu.SemaphoreType.DMA,
        pltpu.SemaphoreType.DMA,
        pltpu.SemaphoreType.DMA,
    ],
)
def _seg_max(h_hbm, src_hbm, dst_hbm, out_hbm,
             dstv, srcv, sbuf, dbuf, rows0, acc, sd0, sd1, sg0):
    _seg_max_body(h_hbm, src_hbm, dst_hbm, out_hbm,
                  dstv, srcv, sbuf, dbuf, rows0, acc, sd0, sd1, sg0)


BR = 2000


def _tc_layer_kernel(h_ref, a_ref, wl_ref, bl_ref, wr_ref, o_ref):
    a = a_ref[...]
    a = jnp.where(jnp.isfinite(a), a, 0.0)
    o = jnp.dot(a, wl_ref[...], preferred_element_type=jnp.float32)
    o = o + jnp.dot(h_ref[...], wr_ref[...], preferred_element_type=jnp.float32)
    o = o + bl_ref[...]
    o_ref[...] = jnp.maximum(o, 0.0)


def _tc_layer(h, aggp, wlt, bl, wrt):
    return pl.pallas_call(
        _tc_layer_kernel,
        grid=(N // BR,),
        in_specs=[
            pl.BlockSpec((BR, D), lambda i: (i, 0)),
            pl.BlockSpec((BR, D), lambda i: (i, 0)),
            pl.BlockSpec((D, D), lambda i: (0, 0)),
            pl.BlockSpec((1, D), lambda i: (0, 0)),
            pl.BlockSpec((D, D), lambda i: (0, 0)),
        ],
        out_specs=pl.BlockSpec((BR, D), lambda i: (i, 0)),
        out_shape=jax.ShapeDtypeStruct((N, D), jnp.float32),
    )(h, aggp, wlt, bl, wrt)


def _tc_final_kernel(h_ref, wt_ref, b_ref, q_ref, o_ref):
    g = jnp.dot(h_ref[...], wt_ref[...], preferred_element_type=jnp.float32)
    g = g + b_ref[...]
    q = q_ref[...]
    qn = jnp.sqrt(jnp.sum(q * q))
    nrm = jnp.sqrt(jnp.sum(g * g, axis=1, keepdims=True))
    s = jnp.dot(g, q.T, preferred_element_type=jnp.float32)
    o_ref[...] = s / (jnp.maximum(nrm, 1e-12) * jnp.maximum(qn, 1e-12))


def _tc_final(h, wlint, blin, q):
    return pl.pallas_call(
        _tc_final_kernel,
        grid=(N // BR,),
        in_specs=[
            pl.BlockSpec((BR, D), lambda i: (i, 0)),
            pl.BlockSpec((D, D), lambda i: (0, 0)),
            pl.BlockSpec((1, D), lambda i: (0, 0)),
            pl.BlockSpec((1, D), lambda i: (0, 0)),
        ],
        out_specs=pl.BlockSpec((BR, 1), lambda i: (i, 0)),
        out_shape=jax.ShapeDtypeStruct((N, 1), jnp.float32),
    )(h, wlint, blin, q)


def kernel(x, edge_index, query,
           W_l0, b_l0, W_r0, W_l1, b_l1, W_r1, W_l2, b_l2, W_r2,
           W_lin, b_lin):
    src = edge_index[0].astype(jnp.int32)
    dst = edge_index[1].astype(jnp.int32)
    params = [(W_l0, b_l0, W_r0), (W_l1, b_l1, W_r1), (W_l2, b_l2, W_r2)]
    h = x
    for (wl, bl, wr) in params:
        aggp = _seg_max(h, src, dst)
        h = _tc_layer(h, aggp[:N], wl.T, bl.reshape(1, D), wr.T)
    scores = _tc_final(h, W_lin.T, b_lin.reshape(1, D), query.reshape(1, D))
    return scores[:, 0]
